# Initial kernel scaffold; baseline (speedup 1.0000x reference)
import jax
import jax.numpy as jnp
from jax.experimental import pallas as pl


def kernel(x, edge_index, ptr, params):
    raise NotImplementedError("write your pallas kernel here")


# speedup vs baseline: 14.1613x; 14.1613x over previous
import functools

import jax
import jax.numpy as jnp
from jax import lax
from jax.experimental import pallas as pl
from jax.experimental.pallas import tpu as pltpu
from jax.experimental.pallas import tpu_sc as plsc

_NC = 2
_NS = 16
_NW = _NC * _NS
_CHUNK = 128


def _sc_mesh():
    return plsc.VectorSubcoreMesh(core_axis_name="c", subcore_axis_name="s")


def _sc_params():
    return pltpu.CompilerParams(use_tc_tiling_on_sc=False)


@functools.lru_cache(maxsize=None)
def _sc_count_kernel(epad, n_rows):
    per_tile = epad // _NW
    n_chunks = per_tile // _CHUNK
    rows_per_tile = n_rows // _NS

    @functools.partial(
        pl.kernel,
        out_type=jax.ShapeDtypeStruct((_NC, n_rows, 16), jnp.float32),
        mesh=_sc_mesh(),
        compiler_params=_sc_params(),
        scratch_types=[
            pltpu.VMEM_SHARED((n_rows, 16), jnp.float32),
            pltpu.VMEM((_CHUNK,), jnp.int32),
            pltpu.VMEM((_CHUNK, 16), jnp.float32),
        ],
    )
    def k(dst_hbm, ones_hbm, zeros_hbm, out_hbm, acc_sh, idx_v, ones_v):
        cid = lax.axis_index("c")
        sid = lax.axis_index("s")
        wid = cid * _NS + sid
        r0 = sid * rows_per_tile
        pltpu.sync_copy(zeros_hbm.at[pl.ds(r0, rows_per_tile)],
                        acc_sh.at[pl.ds(r0, rows_per_tile)])
        pltpu.sync_copy(ones_hbm, ones_v)
        plsc.subcore_barrier()
        base = wid * per_tile

        def body(i, carry):
            s = base + i * _CHUNK
            pltpu.sync_copy(dst_hbm.at[pl.ds(s, _CHUNK)], idx_v)
            pltpu.sync_copy(ones_v, acc_sh.at[idx_v], add=True)
            return carry

        lax.fori_loop(0, n_chunks, body, 0)
        plsc.subcore_barrier()
        pltpu.sync_copy(acc_sh.at[pl.ds(r0, rows_per_tile)],
                        out_hbm.at[cid, pl.ds(r0, rows_per_tile)])

    return k


@functools.lru_cache(maxsize=None)
def _sc_segsum_kernel(epad, n_rows, feat):
    per_tile = epad // _NW
    n_chunks = per_tile // _CHUNK
    rows_per_tile = n_rows // _NS

    @functools.partial(
        pl.kernel,
        out_type=jax.ShapeDtypeStruct((_NC, n_rows, feat), jnp.float32),
        mesh=_sc_mesh(),
        compiler_params=_sc_params(),
        scratch_types=[
            pltpu.VMEM_SHARED((n_rows, feat), jnp.float32),
            pltpu.VMEM_SHARED((n_rows, feat), jnp.float32),
            pltpu.VMEM((_CHUNK,), jnp.int32),
            pltpu.VMEM((_CHUNK,), jnp.int32),
            pltpu.VMEM((_CHUNK, feat), jnp.float32),
            pltpu.SemaphoreType.DMA,
        ],
    )
    def k(hp_hbm, src_hbm, dst_hbm, zeros_hbm, out_hbm,
          table_sh, acc_sh, sidx_v, didx_v, rows_v, sem):
        cid = lax.axis_index("c")
        sid = lax.axis_index("s")
        wid = cid * _NS + sid
        r0 = sid * rows_per_tile
        pltpu.sync_copy(hp_hbm.at[pl.ds(r0, rows_per_tile)],
                        table_sh.at[pl.ds(r0, rows_per_tile)])
        pltpu.sync_copy(zeros_hbm.at[pl.ds(r0, rows_per_tile)],
                        acc_sh.at[pl.ds(r0, rows_per_tile)])
        plsc.subcore_barrier()
        base = wid * per_tile

        def body(i, carry):
            s = base + i * _CHUNK
            pltpu.sync_copy(src_hbm.at[pl.ds(s, _CHUNK)], sidx_v)
            pltpu.async_copy(table_sh.at[sidx_v], rows_v, sem).wait()
            pltpu.sync_copy(dst_hbm.at[pl.ds(s, _CHUNK)], didx_v)
            pltpu.sync_copy(rows_v, acc_sh.at[didx_v], add=True)
            return carry

        lax.fori_loop(0, n_chunks, body, 0)
        plsc.subcore_barrier()
        pltpu.sync_copy(acc_sh.at[pl.ds(r0, rows_per_tile)],
                        out_hbm.at[cid, pl.ds(r0, rows_per_tile)])

    return k


def _tc_deg(cnt, n):
    def body(c_ref, inv_ref, invdeg_ref):
        deg = 1.0 + c_ref[0, :n, 0:1] + c_ref[1, :n, 0:1]
        inv_ref[...] = lax.rsqrt(deg)
        invdeg_ref[...] = 1.0 / deg

    return pl.pallas_call(
        body,
        out_shape=(jax.ShapeDtypeStruct((n, 1), jnp.float32),
                   jax.ShapeDtypeStruct((n, 1), jnp.float32)),
    )(cnt)


def _tc_pre(a, w, inv, n_rows):
    n = a.shape[0]
    f = w.shape[1]

    def body(a_ref, w_ref, inv_ref, h_ref, hp_ref):
        h = jnp.dot(a_ref[...], w_ref[...], preferred_element_type=jnp.float32)
        h_ref[...] = h
        hp_ref[:n] = h * inv_ref[...]
        hp_ref[n:] = jnp.zeros((n_rows - n, f), jnp.float32)

    return pl.pallas_call(
        body,
        out_shape=(jax.ShapeDtypeStruct((n, f), jnp.float32),
                   jax.ShapeDtypeStruct((n_rows, f), jnp.float32)),
    )(a, w, inv)


def _tc_post(p, h, inv, invdeg, b, g, be, res):
    n, f = h.shape

    def body(p_ref, h_ref, inv_ref, invdeg_ref, b_ref, g_ref, be_ref,
             res_ref, o_ref):
        e = (p_ref[0, :n, :] + p_ref[1, :n, :]) * inv_ref[...]
        hh = h_ref[...]
        s = (e + hh * invdeg_ref[...]) * invdeg_ref[...] + b_ref[...]
        t = jnp.where(s >= 0.0, s, 0.01 * s)
        mu = jnp.mean(t, axis=0, keepdims=True)
        var = jnp.mean((t - mu) * (t - mu), axis=0, keepdims=True)
        o_ref[...] = (res_ref[...]
                      + (t - mu) * lax.rsqrt(var + 1e-5) * g_ref[...]
                      + be_ref[...])

    return pl.pallas_call(
        body,
        out_shape=jax.ShapeDtypeStruct((n, f), jnp.float32),
    )(p, h, inv, invdeg, b.reshape(1, f), g.reshape(1, f), be.reshape(1, f),
      res)


def _tc_final(h, wl, bl):
    n = h.shape[0]
    f = wl.shape[1]

    def body(h_ref, w_ref, b_ref, o_ref):
        o_ref[...] = (jnp.dot(h_ref[...], w_ref[...],
                              preferred_element_type=jnp.float32)
                      + b_ref[...])

    return pl.pallas_call(
        body,
        out_shape=jax.ShapeDtypeStruct((n, f), jnp.float32),
    )(h, wl, bl.reshape(1, f))


def kernel(x, edge_index, ptr, params):
    n = x.shape[0]
    n_rows = ((n // (_NS * 8)) + 1) * (_NS * 8)
    e = edge_index.shape[1]
    batch = int(ptr.shape[0]) - 1
    out_ch = params['Wl'].shape[1]

    grain = _NW * _CHUNK
    epad = ((e + grain - 1) // grain) * grain
    pad_cfg = ((0, epad - e),)
    src = jnp.pad(edge_index[0], pad_cfg, constant_values=n)
    dst = jnp.pad(edge_index[1], pad_cfg, constant_values=n)

    ones16 = jnp.ones((_CHUNK, 16), jnp.float32)
    zeros16 = jnp.zeros((n_rows, 16), jnp.float32)

    cnt = _sc_count_kernel(epad, n_rows)(dst, ones16, zeros16)
    inv, invdeg = _tc_deg(cnt, n)

    feats = []
    h = x
    for i in range(4):
        w = params['W%d' % i]
        hm, hp = _tc_pre(h, w, inv, n_rows)
        f = w.shape[1]
        zeros_f = jnp.zeros((n_rows, f), jnp.float32)
        p = _sc_segsum_kernel(epad, n_rows, f)(hp, src, dst, zeros_f)
        h = _tc_post(p, hm, inv, invdeg, params['b%d' % i],
                     params['g%d' % i], params['be%d' % i],
                     jnp.zeros((n, f), jnp.float32))
        feats.append(h)
    for j in range(4):
        i = 4 + j
        w = params['W%d' % i]
        hm, hp = _tc_pre(h, w, inv, n_rows)
        f = w.shape[1]
        zeros_f = jnp.zeros((n_rows, f), jnp.float32)
        p = _sc_segsum_kernel(epad, n_rows, f)(hp, src, dst, zeros_f)
        h = _tc_post(p, hm, inv, invdeg, params['b%d' % i],
                     params['g%d' % i], params['be%d' % i], feats[3 - j])

    y = _tc_final(h, params['Wl'], params['bl'])
    y = y.reshape(batch, n // batch, out_ch)
    return jnp.transpose(y, (0, 2, 1))


# speedup vs baseline: 24.8724x; 1.7564x over previous
import functools

import jax
import jax.numpy as jnp
from jax import lax
from jax.experimental import pallas as pl
from jax.experimental.pallas import tpu as pltpu
from jax.experimental.pallas import tpu_sc as plsc

_NC = 2
_NS = 16
_NW = _NC * _NS
_CHUNK = 128


def _sc_mesh():
    return plsc.VectorSubcoreMesh(core_axis_name="c", subcore_axis_name="s")


def _sc_params():
    return pltpu.CompilerParams(use_tc_tiling_on_sc=False)


@functools.lru_cache(maxsize=None)
def _sc_count_kernel(n_chunks, n_rows):
    rows_per_tile = n_rows // _NS

    @functools.partial(
        pl.kernel,
        out_type=jax.ShapeDtypeStruct((_NC, n_rows, 16), jnp.float32),
        mesh=_sc_mesh(),
        compiler_params=_sc_params(),
        scratch_types=[
            pltpu.VMEM_SHARED((n_rows, 16), jnp.float32),
            pltpu.VMEM((n_chunks, _CHUNK), jnp.int32),
            pltpu.VMEM((_CHUNK, 16), jnp.float32),
        ],
    )
    def k(dst_hbm, ones_hbm, zeros_hbm, out_hbm, acc_sh, didx_v, ones_v):
        cid = lax.axis_index("c")
        sid = lax.axis_index("s")
        wid = cid * _NS + sid
        r0 = sid * rows_per_tile
        pltpu.sync_copy(zeros_hbm.at[pl.ds(r0, rows_per_tile)],
                        acc_sh.at[pl.ds(r0, rows_per_tile)])
        pltpu.sync_copy(ones_hbm, ones_v)
        pltpu.sync_copy(dst_hbm.at[wid], didx_v)
        plsc.subcore_barrier()

        def body(i, carry):
            pltpu.sync_copy(ones_v, acc_sh.at[didx_v.at[i]], add=True)
            return carry

        lax.fori_loop(0, n_chunks, body, 0)
        plsc.subcore_barrier()
        pltpu.sync_copy(acc_sh.at[pl.ds(r0, rows_per_tile)],
                        out_hbm.at[cid, pl.ds(r0, rows_per_tile)])

    return k


@functools.lru_cache(maxsize=None)
def _sc_segsum_kernel(n_chunks, n_rows, feat):
    rows_per_tile = n_rows // _NS

    @functools.partial(
        pl.kernel,
        out_type=jax.ShapeDtypeStruct((_NC, n_rows, feat), jnp.float32),
        mesh=_sc_mesh(),
        compiler_params=_sc_params(),
        scratch_types=[
            pltpu.VMEM_SHARED((n_rows, feat), jnp.float32),
            pltpu.VMEM_SHARED((n_rows, feat), jnp.float32),
            pltpu.VMEM((n_chunks, _CHUNK), jnp.int32),
            pltpu.VMEM((n_chunks, _CHUNK), jnp.int32),
            pltpu.VMEM((_CHUNK, feat), jnp.float32),
            pltpu.VMEM((_CHUNK, feat), jnp.float32),
            pltpu.SemaphoreType.DMA,
            pltpu.SemaphoreType.DMA,
        ],
    )
    def k(hp_hbm, src_hbm, dst_hbm, zeros_hbm, out_hbm,
          table_sh, acc_sh, sidx_v, didx_v, buf0, buf1, sem0, sem1):
        cid = lax.axis_index("c")
        sid = lax.axis_index("s")
        wid = cid * _NS + sid
        r0 = sid * rows_per_tile
        pltpu.sync_copy(hp_hbm.at[pl.ds(r0, rows_per_tile)],
                        table_sh.at[pl.ds(r0, rows_per_tile)])
        pltpu.sync_copy(zeros_hbm.at[pl.ds(r0, rows_per_tile)],
                        acc_sh.at[pl.ds(r0, rows_per_tile)])
        pltpu.sync_copy(src_hbm.at[wid], sidx_v)
        pltpu.sync_copy(dst_hbm.at[wid], didx_v)
        plsc.subcore_barrier()

        pltpu.async_copy(table_sh.at[sidx_v.at[0]], buf0, sem0)

        def pair(j, carry):
            c0 = 2 * j
            pltpu.async_copy(table_sh.at[sidx_v.at[c0 + 1]], buf1, sem1)
            pltpu.make_async_copy(table_sh.at[sidx_v.at[c0]], buf0, sem0).wait()
            pltpu.sync_copy(buf0, acc_sh.at[didx_v.at[c0]], add=True)
            pltpu.async_copy(table_sh.at[sidx_v.at[c0 + 2]], buf0, sem0)
            pltpu.make_async_copy(table_sh.at[sidx_v.at[c0 + 1]], buf1,
                                  sem1).wait()
            pltpu.sync_copy(buf1, acc_sh.at[didx_v.at[c0 + 1]], add=True)
            return carry

        lax.fori_loop(0, (n_chunks - 1) // 2, pair, 0)
        last = n_chunks - 1
        pltpu.make_async_copy(table_sh.at[sidx_v.at[last]], buf0, sem0).wait()
        pltpu.sync_copy(buf0, acc_sh.at[didx_v.at[last]], add=True)
        plsc.subcore_barrier()
        pltpu.sync_copy(acc_sh.at[pl.ds(r0, rows_per_tile)],
                        out_hbm.at[cid, pl.ds(r0, rows_per_tile)])

    return k


def _tc_deg(cnt, n):
    def body(c_ref, inv_ref, invdeg_ref):
        deg = 1.0 + c_ref[0, :n, 0:1] + c_ref[1, :n, 0:1]
        inv_ref[...] = lax.rsqrt(deg)
        invdeg_ref[...] = 1.0 / deg

    return pl.pallas_call(
        body,
        out_shape=(jax.ShapeDtypeStruct((n, 1), jnp.float32),
                   jax.ShapeDtypeStruct((n, 1), jnp.float32)),
    )(cnt)


def _tc_pre(a, w, inv, n_rows):
    n = a.shape[0]
    f = w.shape[1]

    def body(a_ref, w_ref, inv_ref, h_ref, hp_ref):
        h = jnp.dot(a_ref[...], w_ref[...], preferred_element_type=jnp.float32)
        h_ref[...] = h
        hp_ref[:n] = h * inv_ref[...]
        hp_ref[n:] = jnp.zeros((n_rows - n, f), jnp.float32)

    return pl.pallas_call(
        body,
        out_shape=(jax.ShapeDtypeStruct((n, f), jnp.float32),
                   jax.ShapeDtypeStruct((n_rows, f), jnp.float32)),
    )(a, w, inv)


def _tc_post(p, h, inv, invdeg, b, g, be, res):
    n, f = h.shape

    def body(p_ref, h_ref, inv_ref, invdeg_ref, b_ref, g_ref, be_ref,
             res_ref, o_ref):
        e = (p_ref[0, :n, :] + p_ref[1, :n, :]) * inv_ref[...]
        hh = h_ref[...]
        s = (e + hh * invdeg_ref[...]) * invdeg_ref[...] + b_ref[...]
        t = jnp.where(s >= 0.0, s, 0.01 * s)
        mu = jnp.mean(t, axis=0, keepdims=True)
        var = jnp.mean((t - mu) * (t - mu), axis=0, keepdims=True)
        o_ref[...] = (res_ref[...]
                      + (t - mu) * lax.rsqrt(var + 1e-5) * g_ref[...]
                      + be_ref[...])

    return pl.pallas_call(
        body,
        out_shape=jax.ShapeDtypeStruct((n, f), jnp.float32),
    )(p, h, inv, invdeg, b.reshape(1, f), g.reshape(1, f), be.reshape(1, f),
      res)


def _tc_final(h, wl, bl):
    n = h.shape[0]
    f = wl.shape[1]

    def body(h_ref, w_ref, b_ref, o_ref):
        o_ref[...] = (jnp.dot(h_ref[...], w_ref[...],
                              preferred_element_type=jnp.float32)
                      + b_ref[...])

    return pl.pallas_call(
        body,
        out_shape=jax.ShapeDtypeStruct((n, f), jnp.float32),
    )(h, wl, bl.reshape(1, f))


def kernel(x, edge_index, ptr, params):
    n = x.shape[0]
    n_rows = ((n // (_NS * 8)) + 1) * (_NS * 8)
    e = edge_index.shape[1]
    batch = int(ptr.shape[0]) - 1
    out_ch = params['Wl'].shape[1]

    grain = _NW * _CHUNK
    n_chunks = max(1, (e + grain - 1) // grain)
    if n_chunks % 2 == 0:
        n_chunks += 1
    epad = n_chunks * grain
    pad_cfg = ((0, epad - e),)
    idx_shape = (_NW, n_chunks, _CHUNK)
    src = jnp.pad(edge_index[0], pad_cfg, constant_values=n).reshape(idx_shape)
    dst = jnp.pad(edge_index[1], pad_cfg, constant_values=n).reshape(idx_shape)

    ones16 = jnp.ones((_CHUNK, 16), jnp.float32)
    zeros16 = jnp.zeros((n_rows, 16), jnp.float32)

    cnt = _sc_count_kernel(n_chunks, n_rows)(dst, ones16, zeros16)
    inv, invdeg = _tc_deg(cnt, n)

    feats = []
    h = x
    for i in range(4):
        w = params['W%d' % i]
        hm, hp = _tc_pre(h, w, inv, n_rows)
        f = w.shape[1]
        zeros_f = jnp.zeros((n_rows, f), jnp.float32)
        p = _sc_segsum_kernel(n_chunks, n_rows, f)(hp, src, dst, zeros_f)
        h = _tc_post(p, hm, inv, invdeg, params['b%d' % i],
                     params['g%d' % i], params['be%d' % i],
                     jnp.zeros((n, f), jnp.float32))
        feats.append(h)
    for j in range(4):
        i = 4 + j
        w = params['W%d' % i]
        hm, hp = _tc_pre(h, w, inv, n_rows)
        f = w.shape[1]
        zeros_f = jnp.zeros((n_rows, f), jnp.float32)
        p = _sc_segsum_kernel(n_chunks, n_rows, f)(hp, src, dst, zeros_f)
        h = _tc_post(p, hm, inv, invdeg, params['b%d' % i],
                     params['g%d' % i], params['be%d' % i], feats[3 - j])

    y = _tc_final(h, params['Wl'], params['bl'])
    y = y.reshape(batch, n // batch, out_ch)
    return jnp.transpose(y, (0, 2, 1))
